# baseline (speedup 1.0000x reference)
import functools

import jax
import jax.numpy as jnp
from jax import lax
from jax.experimental import pallas as pl
from jax.experimental.pallas import tpu as pltpu
from jax.experimental.pallas import tpu_sc as plsc

B = 16384
OUT_DIM = 128
LUT = 512
NC, NS = 2, 16
NW = NC * NS
BPW = B // NW
IG = 4
CH = BPW // IG


def _table_kernel(dir_ref, ext_ref, typ_ref, w_ref, b_ref, ft_ref,
                  lut_ref, c_ref):
    ft = ft_ref[...]
    c2 = ft[0:1, :] * 64 + ft[1:2, :] * 8 + ft[2:3, :]
    c_ref[...] = c2.reshape(B)
    w = w_ref[...]
    pd = lax.dot_general(
        dir_ref[:, 0:8], w[0:5, :], (((0,), (0,)), ((), ())),
        preferred_element_type=jnp.float32,
    )
    pe7 = jnp.dot(ext_ref[...], w[5:10, :], preferred_element_type=jnp.float32)
    pe = jnp.concatenate([pe7, jnp.zeros((1, OUT_DIM), jnp.float32)], 0)
    pt = jnp.dot(typ_ref[...], w[10:15, :], preferred_element_type=jnp.float32)
    pet = (pe[:, None, :] + pt[None, :, :]).reshape(64, OUT_DIM)
    full = (pd[:, None, :] + pet[None, :, :]).reshape(LUT, OUT_DIM)
    lut_ref[...] = jax.nn.sigmoid(full + b_ref[...][None, :])


def _build_table(dir_t, ext_t, typ_t, w, b, ft):
    return pl.pallas_call(
        _table_kernel,
        out_shape=[
            jax.ShapeDtypeStruct((LUT, OUT_DIM), jnp.float32),
            jax.ShapeDtypeStruct((B,), jnp.int32),
        ],
    )(dir_t, ext_t, typ_t, w, b, ft)


SL = LUT // NS


def _sc_gather_body(cidx_hbm, lut_hbm, out_hbm, cidx, rows_v, lut_sp, *sems):
    gsems, wsem, csem = sems[:IG], sems[IG], sems[IG + 1]
    sid = lax.axis_index("s")
    wid = sid * NC + lax.axis_index("c")
    base = wid * BPW

    ccp = pltpu.async_copy(cidx_hbm.at[pl.ds(base, BPW)], cidx, csem)
    sl = pl.ds(sid * SL, SL)
    pltpu.sync_copy(lut_hbm.at[sl], lut_sp.at[sl])
    plsc.subcore_barrier()
    ccp.wait()
    gcp = [
        pltpu.async_copy(
            lut_sp.at[cidx.at[pl.ds(g * CH, CH)]], rows_v.at[g], gsems[g]
        )
        for g in range(IG)
    ]
    wcp = []
    for g in range(IG):
        gcp[g].wait()
        wcp.append(
            pltpu.async_copy(
                rows_v.at[g], out_hbm.at[pl.ds(base + g * CH, CH)], wsem
            )
        )
    for cp in wcp:
        cp.wait()


@functools.cache
def _sc_gather():
    return pl.kernel(
        _sc_gather_body,
        mesh=plsc.VectorSubcoreMesh(
            core_axis_name="c",
            subcore_axis_name="s",
            num_cores=NC,
            num_subcores=NS,
        ),
        out_type=jax.ShapeDtypeStruct((B, OUT_DIM), jnp.float32),
        scratch_types=[
            pltpu.VMEM((BPW,), jnp.int32),
            pltpu.VMEM((IG, CH, OUT_DIM), jnp.float32),
            pltpu.VMEM_SHARED((LUT, OUT_DIM), jnp.float32),
        ]
        + [pltpu.SemaphoreType.DMA] * (IG + 2),
        compiler_params=pltpu.CompilerParams(needs_layout_passes=False),
    )


def kernel(features, dir_table, ext_table, type_table, W, b):
    ft = features.astype(jnp.int32).T
    lut, c = _build_table(dir_table.T, ext_table, type_table, W, b, ft)
    return _sc_gather()(c, lut)


# speedup vs baseline: 1.0007x; 1.0007x over previous
import functools

import jax
import jax.numpy as jnp
from jax import lax
from jax.experimental import pallas as pl
from jax.experimental.pallas import tpu as pltpu
from jax.experimental.pallas import tpu_sc as plsc

B = 16384
OUT_DIM = 128
LUT = 512
NC, NS = 2, 16
NW = NC * NS
BPW = B // NW
IG = 16
CH = BPW // IG


def _table_kernel(dir_ref, ext_ref, typ_ref, w_ref, b_ref, ft_ref,
                  lut_ref, c_ref):
    ft = ft_ref[...]
    c2 = ft[0:1, :] * 64 + ft[1:2, :] * 8 + ft[2:3, :]
    c_ref[...] = c2.reshape(B)
    w = w_ref[...]
    pd = lax.dot_general(
        dir_ref[:, 0:8], w[0:5, :], (((0,), (0,)), ((), ())),
        preferred_element_type=jnp.float32,
    )
    pe7 = jnp.dot(ext_ref[...], w[5:10, :], preferred_element_type=jnp.float32)
    pe = jnp.concatenate([pe7, jnp.zeros((1, OUT_DIM), jnp.float32)], 0)
    pt = jnp.dot(typ_ref[...], w[10:15, :], preferred_element_type=jnp.float32)
    pet = (pe[:, None, :] + pt[None, :, :]).reshape(64, OUT_DIM)
    full = (pd[:, None, :] + pet[None, :, :]).reshape(LUT, OUT_DIM)
    lut_ref[...] = jax.nn.sigmoid(full + b_ref[...][None, :])


def _build_table(dir_t, ext_t, typ_t, w, b, ft):
    return pl.pallas_call(
        _table_kernel,
        out_shape=[
            jax.ShapeDtypeStruct((LUT, OUT_DIM), jnp.float32),
            jax.ShapeDtypeStruct((B,), jnp.int32),
        ],
    )(dir_t, ext_t, typ_t, w, b, ft)


SL = LUT // NS


def _sc_gather_body(cidx_hbm, lut_hbm, out_hbm, cidx, rows_v, lut_sp, *sems):
    gsems, wsem, csem = sems[:IG], sems[IG], sems[IG + 1]
    sid = lax.axis_index("s")
    wid = sid * NC + lax.axis_index("c")
    base = wid * BPW

    ccp = pltpu.async_copy(cidx_hbm.at[pl.ds(base, BPW)], cidx, csem)
    sl = pl.ds(sid * SL, SL)
    pltpu.sync_copy(lut_hbm.at[sl], lut_sp.at[sl])
    plsc.subcore_barrier()
    ccp.wait()
    gcp = [
        pltpu.async_copy(
            lut_sp.at[cidx.at[pl.ds(g * CH, CH)]], rows_v.at[g], gsems[g]
        )
        for g in range(IG)
    ]
    wcp = []
    for g in range(IG):
        gcp[g].wait()
        wcp.append(
            pltpu.async_copy(
                rows_v.at[g], out_hbm.at[pl.ds(base + g * CH, CH)], wsem
            )
        )
    for cp in wcp:
        cp.wait()


@functools.cache
def _sc_gather():
    return pl.kernel(
        _sc_gather_body,
        mesh=plsc.VectorSubcoreMesh(
            core_axis_name="c",
            subcore_axis_name="s",
            num_cores=NC,
            num_subcores=NS,
        ),
        out_type=jax.ShapeDtypeStruct((B, OUT_DIM), jnp.float32),
        scratch_types=[
            pltpu.VMEM((BPW,), jnp.int32),
            pltpu.VMEM((IG, CH, OUT_DIM), jnp.float32),
            pltpu.VMEM_SHARED((LUT, OUT_DIM), jnp.float32),
        ]
        + [pltpu.SemaphoreType.DMA] * (IG + 2),
        compiler_params=pltpu.CompilerParams(needs_layout_passes=False),
    )


def kernel(features, dir_table, ext_table, type_table, W, b):
    ft = features.astype(jnp.int32).T
    lut, c = _build_table(dir_table.T, ext_table, type_table, W, b, ft)
    return _sc_gather()(c, lut)


# speedup vs baseline: 1.0051x; 1.0043x over previous
import functools

import jax
import jax.numpy as jnp
from jax import lax
from jax.experimental import pallas as pl
from jax.experimental.pallas import tpu as pltpu
from jax.experimental.pallas import tpu_sc as plsc

B = 16384
OUT_DIM = 128
LUT = 512
NC, NS = 2, 16
NW = NC * NS
BPW = B // NW
IG = 8
CH = BPW // IG


def _table_kernel(dir_ref, ext_ref, typ_ref, w_ref, b_ref, ft_ref,
                  lut_ref, c_ref):
    ft = ft_ref[...]
    c2 = ft[0:1, :] * 64 + ft[1:2, :] * 8 + ft[2:3, :]
    c_ref[...] = c2.reshape(B)
    w = w_ref[...]
    pd = lax.dot_general(
        dir_ref[:, 0:8], w[0:5, :], (((0,), (0,)), ((), ())),
        preferred_element_type=jnp.float32,
    )
    pe7 = jnp.dot(ext_ref[...], w[5:10, :], preferred_element_type=jnp.float32)
    pe = jnp.concatenate([pe7, jnp.zeros((1, OUT_DIM), jnp.float32)], 0)
    pt = jnp.dot(typ_ref[...], w[10:15, :], preferred_element_type=jnp.float32)
    pet = (pe[:, None, :] + pt[None, :, :]).reshape(64, OUT_DIM)
    full = (pd[:, None, :] + pet[None, :, :]).reshape(LUT, OUT_DIM)
    lut_ref[...] = jax.nn.sigmoid(full + b_ref[...][None, :])


def _build_table(dir_t, ext_t, typ_t, w, b, ft):
    return pl.pallas_call(
        _table_kernel,
        out_shape=[
            jax.ShapeDtypeStruct((LUT, OUT_DIM), jnp.float32),
            jax.ShapeDtypeStruct((B,), jnp.int32),
        ],
    )(dir_t, ext_t, typ_t, w, b, ft)


SL = LUT // NS


def _sc_gather_body(cidx_hbm, lut_hbm, out_hbm, cidx, rows_v, lut_sp, *sems):
    gsems, wsem, csem = sems[:IG], sems[IG], sems[IG + 1]
    sid = lax.axis_index("s")
    wid = sid * NC + lax.axis_index("c")
    base = wid * BPW

    ccp = pltpu.async_copy(cidx_hbm.at[pl.ds(base, BPW)], cidx, csem)
    sl = pl.ds(sid * SL, SL)
    pltpu.sync_copy(lut_hbm.at[sl], lut_sp.at[sl])
    plsc.subcore_barrier()
    ccp.wait()
    gcp = [
        pltpu.async_copy(
            lut_sp.at[cidx.at[pl.ds(g * CH, CH)]], rows_v.at[g], gsems[g]
        )
        for g in range(IG)
    ]
    wcp = []
    for g in range(IG):
        gcp[g].wait()
        wcp.append(
            pltpu.async_copy(
                rows_v.at[g], out_hbm.at[pl.ds(base + g * CH, CH)], wsem
            )
        )
    for cp in wcp:
        cp.wait()


@functools.cache
def _sc_gather():
    return pl.kernel(
        _sc_gather_body,
        mesh=plsc.VectorSubcoreMesh(
            core_axis_name="c",
            subcore_axis_name="s",
            num_cores=NC,
            num_subcores=NS,
        ),
        out_type=jax.ShapeDtypeStruct((B, OUT_DIM), jnp.float32),
        scratch_types=[
            pltpu.VMEM((BPW,), jnp.int32),
            pltpu.VMEM((IG, CH, OUT_DIM), jnp.float32),
            pltpu.VMEM_SHARED((LUT, OUT_DIM), jnp.float32),
        ]
        + [pltpu.SemaphoreType.DMA] * (IG + 2),
        compiler_params=pltpu.CompilerParams(needs_layout_passes=False),
    )


def kernel(features, dir_table, ext_table, type_table, W, b):
    ft = features.astype(jnp.int32).T
    lut, c = _build_table(dir_table.T, ext_table, type_table, W, b, ft)
    return _sc_gather()(c, lut)
